# Initial kernel scaffold; baseline (speedup 1.0000x reference)
import functools
import jax
import jax.numpy as jnp
from jax import lax
from jax.experimental import pallas as pl
from jax.experimental.pallas import tpu as pltpu
from jax.experimental.pallas import tpu_sc as plsc

DIM_POS = 64
DIM_TOK = 128
DIM_OUT = DIM_POS + DIM_TOK

_NC = 2
_NS = 16
_NW = _NC * _NS

_CHUNK = 128


def _make_kernel(n_flat):
    assert n_flat % (_NW * _CHUNK) == 0
    bpw = n_flat // _NW
    n_chunks = bpw // _CHUNK
    mesh = plsc.VectorSubcoreMesh(core_axis_name="c", subcore_axis_name="s")

    @functools.partial(
        pl.kernel,
        out_type=jax.ShapeDtypeStruct((n_flat, DIM_OUT), jnp.float32),
        mesh=mesh,
        scratch_types=[
            pltpu.VMEM((_CHUNK,), jnp.int32),
            pltpu.VMEM((_CHUNK,), jnp.int32),
            pltpu.VMEM((_CHUNK, DIM_TOK), jnp.float32),
            pltpu.VMEM((_CHUNK, DIM_POS), jnp.float32),
            pltpu.SemaphoreType.DMA,
        ],
    )
    def embed(tok_hbm, pos_hbm, wt_hbm, wp_hbm, out_hbm,
              tok_idx, pos_idx, tok_rows, pos_rows, sem):
        wid = lax.axis_index("s") * _NC + lax.axis_index("c")
        base = wid * bpw

        def body(i, carry):
            off = base + i * _CHUNK
            pltpu.sync_copy(tok_hbm.at[pl.ds(off, _CHUNK)], tok_idx)
            pltpu.sync_copy(pos_hbm.at[pl.ds(off, _CHUNK)], pos_idx)
            pltpu.async_copy(wt_hbm.at[tok_idx], tok_rows, sem).wait()
            pltpu.async_copy(wp_hbm.at[pos_idx], pos_rows, sem).wait()
            pltpu.sync_copy(pos_rows,
                            out_hbm.at[pl.ds(off, _CHUNK), pl.ds(0, DIM_POS)])
            pltpu.sync_copy(tok_rows,
                            out_hbm.at[pl.ds(off, _CHUNK), pl.ds(DIM_POS, DIM_TOK)])
            return carry

        lax.fori_loop(0, n_chunks, body, 0)

    return embed


def kernel(tokens, pos, W_tokens, W_pos):
    batch, seq = tokens.shape
    n_flat = batch * seq
    tok_flat = tokens.reshape(n_flat).astype(jnp.int32)
    pos_flat = pos.reshape(n_flat).astype(jnp.int32)
    out = _make_kernel(n_flat)(tok_flat, pos_flat, W_tokens, W_pos)
    return out.reshape(batch, seq, DIM_OUT)


# speedup vs baseline: 4.4360x; 4.4360x over previous
import functools
import jax
import jax.numpy as jnp
from jax import lax
from jax.experimental import pallas as pl
from jax.experimental.pallas import tpu as pltpu
from jax.experimental.pallas import tpu_sc as plsc

DIM_POS = 64
DIM_TOK = 128
DIM_OUT = DIM_POS + DIM_TOK

_NC = 2
_NS = 16
_NW = _NC * _NS
_CHUNK = 128


def _make_kernel(n_flat):
    assert n_flat % (_NW * _CHUNK) == 0
    bpw = n_flat // _NW
    n_chunks = bpw // _CHUNK
    assert n_chunks % 2 == 0
    n_pairs = n_chunks // 2
    mesh = plsc.VectorSubcoreMesh(core_axis_name="c", subcore_axis_name="s")

    @functools.partial(
        pl.kernel,
        out_type=jax.ShapeDtypeStruct((n_flat, DIM_OUT), jnp.float32),
        mesh=mesh,
        compiler_params=pltpu.CompilerParams(use_tc_tiling_on_sc=False),
        scratch_types=[
            pltpu.VMEM((n_chunks, _CHUNK), jnp.int32),
            pltpu.VMEM((n_chunks, _CHUNK), jnp.int32),
            pltpu.VMEM((_CHUNK, DIM_TOK), jnp.float32),
            pltpu.VMEM((_CHUNK, DIM_TOK), jnp.float32),
            pltpu.VMEM((_CHUNK, DIM_POS), jnp.float32),
            pltpu.VMEM((_CHUNK, DIM_POS), jnp.float32),
            pltpu.SemaphoreType.DMA,
            pltpu.SemaphoreType.DMA,
            pltpu.SemaphoreType.DMA,
            pltpu.SemaphoreType.DMA,
        ],
    )
    def embed(tok_hbm, pos_hbm, wt_hbm, wp_hbm, out_hbm,
              tok_idx, pos_idx, tok0, tok1, pos0, pos1,
              gsem0, gsem1, osem0, osem1):
        wid = lax.axis_index("s") * _NC + lax.axis_index("c")
        base = wid * bpw
        row0 = wid * n_chunks

        pltpu.sync_copy(tok_hbm.at[pl.ds(row0, n_chunks)], tok_idx)
        pltpu.sync_copy(pos_hbm.at[pl.ds(row0, n_chunks)], pos_idx)

        def issue_gather(g, tok_buf, pos_buf, gsem):
            pltpu.async_copy(wt_hbm.at[tok_idx.at[g]], tok_buf, gsem)
            pltpu.async_copy(wp_hbm.at[pos_idx.at[g]], pos_buf, gsem)

        def drain_gather(tok_buf, pos_buf, gsem):
            pltpu.make_async_copy(wt_hbm.at[pl.ds(0, _CHUNK)], tok_buf, gsem).wait()
            pltpu.make_async_copy(wp_hbm.at[pl.ds(0, _CHUNK)], pos_buf, gsem).wait()

        def issue_out(g, tok_buf, pos_buf, osem):
            off = base + g * _CHUNK
            pltpu.async_copy(
                pos_buf, out_hbm.at[pl.ds(off, _CHUNK), pl.ds(0, DIM_POS)], osem)
            pltpu.async_copy(
                tok_buf, out_hbm.at[pl.ds(off, _CHUNK), pl.ds(DIM_POS, DIM_TOK)], osem)

        def drain_out(tok_buf, pos_buf, osem):
            pltpu.make_async_copy(
                pos_buf, out_hbm.at[pl.ds(0, _CHUNK), pl.ds(0, DIM_POS)], osem).wait()
            pltpu.make_async_copy(
                tok_buf, out_hbm.at[pl.ds(0, _CHUNK), pl.ds(DIM_POS, DIM_TOK)], osem).wait()

        issue_gather(0, tok0, pos0, gsem0)
        issue_gather(1, tok1, pos1, gsem1)

        def body(j, carry):
            g0 = 2 * j
            g1 = 2 * j + 1
            drain_gather(tok0, pos0, gsem0)
            issue_out(g0, tok0, pos0, osem0)
            drain_gather(tok1, pos1, gsem1)
            issue_out(g1, tok1, pos1, osem1)

            @pl.when(j + 1 < n_pairs)
            def _():
                drain_out(tok0, pos0, osem0)
                issue_gather(g0 + 2, tok0, pos0, gsem0)
                drain_out(tok1, pos1, osem1)
                issue_gather(g1 + 2, tok1, pos1, gsem1)

            return carry

        lax.fori_loop(0, n_pairs, body, 0)
        drain_out(tok0, pos0, osem0)
        drain_out(tok1, pos1, osem1)

    return embed


def kernel(tokens, pos, W_tokens, W_pos):
    batch, seq = tokens.shape
    n_flat = batch * seq
    tok_2d = tokens.reshape(n_flat // _CHUNK, _CHUNK).astype(jnp.int32)
    pos_2d = pos.reshape(n_flat // _CHUNK, _CHUNK).astype(jnp.int32)
    out = _make_kernel(n_flat)(tok_2d, pos_2d, W_tokens, W_pos)
    return out.reshape(batch, seq, DIM_OUT)


# speedup vs baseline: 4.4765x; 1.0091x over previous
import functools
import jax
import jax.numpy as jnp
from jax import lax
from jax.experimental import pallas as pl
from jax.experimental.pallas import tpu as pltpu
from jax.experimental.pallas import tpu_sc as plsc

DIM_POS = 64
DIM_TOK = 128
DIM_OUT = DIM_POS + DIM_TOK

_NC = 2
_NS = 16
_NW = _NC * _NS
_NSLOT = 4


def _make_kernel(batch, seq):
    assert batch % _NW == 0
    bpw = batch // _NW
    n_iter = bpw + _NSLOT
    mesh = plsc.VectorSubcoreMesh(core_axis_name="c", subcore_axis_name="s")

    @functools.partial(
        pl.kernel,
        out_type=jax.ShapeDtypeStruct((batch, seq, DIM_OUT), jnp.float32),
        mesh=mesh,
        compiler_params=pltpu.CompilerParams(use_tc_tiling_on_sc=False),
        scratch_types=[
            pltpu.VMEM((bpw, seq), jnp.int32),
            pltpu.VMEM((bpw, seq), jnp.int32),
            [pltpu.VMEM((seq, DIM_TOK), jnp.float32) for _ in range(_NSLOT)],
            [pltpu.VMEM((seq, DIM_POS), jnp.float32) for _ in range(_NSLOT)],
            [pltpu.SemaphoreType.DMA for _ in range(_NSLOT)],
            [pltpu.SemaphoreType.DMA for _ in range(_NSLOT)],
        ],
    )
    def embed(tok_hbm, pos_hbm, wt_hbm, wp_hbm, out_hbm,
              tok_idx, pos_idx, tok_bufs, pos_bufs, gsems, osems):
        wid = lax.axis_index("s") * _NC + lax.axis_index("c")
        b0 = wid * bpw

        pltpu.sync_copy(tok_hbm.at[pl.ds(b0, bpw)], tok_idx)
        pltpu.sync_copy(pos_hbm.at[pl.ds(b0, bpw)], pos_idx)

        def issue_gather(g, s):
            pltpu.async_copy(wt_hbm.at[tok_idx.at[g]], tok_bufs[s], gsems[s])
            pltpu.async_copy(wp_hbm.at[pos_idx.at[g]], pos_bufs[s], gsems[s])

        def drain_gather(s):
            pltpu.make_async_copy(wt_hbm.at[pl.ds(0, seq)], tok_bufs[s],
                                  gsems[s]).wait()
            pltpu.make_async_copy(wp_hbm.at[pl.ds(0, seq)], pos_bufs[s],
                                  gsems[s]).wait()

        def issue_out(g, s):
            b = b0 + g
            pltpu.async_copy(pos_bufs[s],
                             out_hbm.at[b, :, pl.ds(0, DIM_POS)], osems[s])
            pltpu.async_copy(tok_bufs[s],
                             out_hbm.at[b, :, pl.ds(DIM_POS, DIM_TOK)], osems[s])

        def drain_out(s):
            pltpu.make_async_copy(pos_bufs[s],
                                  out_hbm.at[0, :, pl.ds(0, DIM_POS)],
                                  osems[s]).wait()
            pltpu.make_async_copy(tok_bufs[s],
                                  out_hbm.at[0, :, pl.ds(DIM_POS, DIM_TOK)],
                                  osems[s]).wait()

        def body(j, carry):
            for k in range(_NSLOT):
                i = j * _NSLOT + k

                @pl.when((i >= _NSLOT) & (i < bpw + _NSLOT))
                def _(i=i, k=k):
                    drain_out(k)

                @pl.when(i < bpw)
                def _(i=i, k=k):
                    issue_gather(i, k)

                @pl.when((i >= 2) & (i < bpw + 2))
                def _(i=i, k=k):
                    s = (k + _NSLOT - 2) % _NSLOT
                    drain_gather(s)
                    issue_out(i - 2, s)

            return carry

        lax.fori_loop(0, (n_iter + _NSLOT - 1) // _NSLOT, body, 0)

    return embed


def kernel(tokens, pos, W_tokens, W_pos):
    batch, seq = tokens.shape
    return _make_kernel(batch, seq)(
        tokens.astype(jnp.int32), pos.astype(jnp.int32), W_tokens, W_pos)


# speedup vs baseline: 5.5321x; 1.2358x over previous
import functools
import jax
import jax.numpy as jnp
from jax import lax
from jax.experimental import pallas as pl
from jax.experimental.pallas import tpu as pltpu
from jax.experimental.pallas import tpu_sc as plsc

DIM_POS = 64
DIM_TOK = 128
DIM_OUT = DIM_POS + DIM_TOK

_NC = 2
_NS = 16
_NW = _NC * _NS
_NSLOT = 4
_LANES = 16


def _make_kernel(batch, seq):
    assert batch % _NW == 0
    bpw = batch // _NW
    n_iter = bpw + _NSLOT
    mesh = plsc.VectorSubcoreMesh(core_axis_name="c", subcore_axis_name="s")

    @functools.partial(
        pl.kernel,
        out_type=jax.ShapeDtypeStruct((batch, seq, DIM_OUT), jnp.float32),
        mesh=mesh,
        scratch_types=[
            pltpu.VMEM((bpw, seq), jnp.int32),
            pltpu.VMEM((bpw, seq), jnp.int32),
            [pltpu.VMEM((seq, DIM_OUT), jnp.float32) for _ in range(_NSLOT)],
            [pltpu.VMEM((seq, DIM_TOK), jnp.float32) for _ in range(_NSLOT)],
            [pltpu.SemaphoreType.DMA for _ in range(_NSLOT)],
            [pltpu.SemaphoreType.DMA for _ in range(_NSLOT)],
        ],
    )
    def embed(tok_hbm, pos_hbm, wt_hbm, wp_hbm, out_hbm,
              tok_idx, pos_idx, comb_bufs, tok_bufs, gsems, osems):
        wid = lax.axis_index("s") * _NC + lax.axis_index("c")
        b0 = wid * bpw

        pltpu.sync_copy(tok_hbm.at[pl.ds(b0, bpw)], tok_idx)
        pltpu.sync_copy(pos_hbm.at[pl.ds(b0, bpw)], pos_idx)

        def issue_gather(g, s):
            pltpu.async_copy(wp_hbm.at[pos_idx.at[g]],
                             comb_bufs[s].at[:, pl.ds(0, DIM_TOK)], gsems[s])
            pltpu.async_copy(wt_hbm.at[tok_idx.at[g]], tok_bufs[s], gsems[s])

        def drain_gather(g, s):
            pltpu.make_async_copy(wp_hbm.at[pos_idx.at[g]],
                                  comb_bufs[s].at[:, pl.ds(0, DIM_TOK)],
                                  gsems[s]).wait()
            pltpu.make_async_copy(wt_hbm.at[tok_idx.at[g]], tok_bufs[s],
                                  gsems[s]).wait()

        def fixup(s):
            comb = comb_bufs[s]
            tokb = tok_bufs[s]

            def row(r, carry):
                for c in range(DIM_TOK // _LANES):
                    comb[r, pl.ds(DIM_POS + c * _LANES, _LANES)] = (
                        tokb[r, pl.ds(c * _LANES, _LANES)])
                return carry

            lax.fori_loop(0, seq, row, 0)

        def issue_out(g, s):
            pltpu.async_copy(comb_bufs[s], out_hbm.at[b0 + g], osems[s])

        def drain_out(s):
            pltpu.make_async_copy(comb_bufs[s], out_hbm.at[0], osems[s]).wait()

        def body(j, carry):
            for k in range(_NSLOT):
                i = j * _NSLOT + k

                @pl.when((i >= _NSLOT) & (i < bpw + _NSLOT))
                def _(i=i, k=k):
                    drain_out(k)

                @pl.when(i < bpw)
                def _(i=i, k=k):
                    issue_gather(i, k)

                @pl.when((i >= 2) & (i < bpw + 2))
                def _(i=i, k=k):
                    s = (k + _NSLOT - 2) % _NSLOT
                    drain_gather(i - 2, s)
                    fixup(s)
                    issue_out(i - 2, s)

            return carry

        lax.fori_loop(0, (n_iter + _NSLOT - 1) // _NSLOT, body, 0)

    return embed


def kernel(tokens, pos, W_tokens, W_pos):
    batch, seq = tokens.shape
    wp_pad = jnp.pad(W_pos, ((0, 0), (0, DIM_TOK - DIM_POS)))
    return _make_kernel(batch, seq)(
        tokens.astype(jnp.int32), pos.astype(jnp.int32), W_tokens, wp_pad)
